# baseline (device time: 126221 ns/iter reference)
import jax
import jax.numpy as jnp
from jax import lax
from jax.experimental import pallas as pl
from jax.experimental.pallas import tpu as pltpu

BLKV = 1024


def kernel(x, W, labels):
    T, D = x.shape
    _, V_shard = W.shape
    nsteps = V_shard // BLKV

    def body(x_ref, w_ref, lab_ref, out_ref,
             m_ref, s_ref, ll_ref, csend, crecv, send_sem, recv_sem):
        i = pl.program_id(0)
        my_x = lax.axis_index("x")
        my_y = lax.axis_index("y")

        @pl.when(i == 0)
        def _():
            m_ref[...] = jnp.full((T, 1), -jnp.inf, jnp.float32)
            s_ref[...] = jnp.zeros((T, 1), jnp.float32)
            ll_ref[...] = jnp.zeros((T, 1), jnp.float32)

        w = w_ref[...].astype(jnp.bfloat16)
        logits = jnp.dot(x_ref[...], w, preferred_element_type=jnp.float32)

        m_old = m_ref[...]
        m_new = jnp.maximum(m_old, jnp.max(logits, axis=1, keepdims=True))
        s_ref[...] = (s_ref[...] * jnp.exp(m_old - m_new)
                      + jnp.sum(jnp.exp(logits - m_new), axis=1, keepdims=True))
        m_ref[...] = m_new

        local = lab_ref[...] - (my_y * V_shard + i * BLKV)
        cols = lax.broadcasted_iota(jnp.int32, (T, BLKV), 1)
        ll_ref[...] += jnp.sum(jnp.where(cols == local, logits, 0.0),
                               axis=1, keepdims=True)

        @pl.when(i == nsteps - 1)
        def _():
            lse_loc = m_ref[...] + jnp.log(s_ref[...])
            csend[:, 0:1] = lse_loc
            csend[:, 1:2] = ll_ref[...]

            barrier = pltpu.get_barrier_semaphore()
            pl.semaphore_signal(barrier, inc=1,
                                device_id=(my_x, 1 - my_y),
                                device_id_type=pl.DeviceIdType.MESH)
            pl.semaphore_wait(barrier, 1)

            rdma = pltpu.make_async_remote_copy(
                src_ref=csend, dst_ref=crecv,
                send_sem=send_sem, recv_sem=recv_sem,
                device_id=(my_x, 1 - my_y),
                device_id_type=pl.DeviceIdType.MESH,
            )
            rdma.start()
            rdma.wait()

            lse_peer = crecv[:, 0:1]
            ll_peer = crecv[:, 1:2]
            m = jnp.maximum(lse_loc, lse_peer)
            lse_g = m + jnp.log(jnp.exp(lse_loc - m) + jnp.exp(lse_peer - m))
            out_ref[...] = lse_g - (ll_ref[...] + ll_peer)

    out = pl.pallas_call(
        body,
        grid=(nsteps,),
        out_shape=jax.ShapeDtypeStruct((T, 1), jnp.float32),
        in_specs=[
            pl.BlockSpec((T, D), lambda i: (0, 0)),
            pl.BlockSpec((D, BLKV), lambda i: (0, i)),
            pl.BlockSpec((T, 1), lambda i: (0, 0)),
        ],
        out_specs=pl.BlockSpec((T, 1), lambda i: (0, 0)),
        scratch_shapes=[
            pltpu.VMEM((T, 1), jnp.float32),
            pltpu.VMEM((T, 1), jnp.float32),
            pltpu.VMEM((T, 1), jnp.float32),
            pltpu.VMEM((T, 2), jnp.float32),
            pltpu.VMEM((T, 2), jnp.float32),
            pltpu.SemaphoreType.DMA,
            pltpu.SemaphoreType.DMA,
        ],
        compiler_params=pltpu.CompilerParams(
            collective_id=0,
            dimension_semantics=("arbitrary",),
        ),
    )(x.astype(jnp.bfloat16), W, labels.reshape(T, 1))
    return out.reshape(T)
